# baseline (speedup 1.0000x reference)
import jax
import jax.numpy as jnp
from jax import lax
from jax.experimental import pallas as pl
from jax.experimental.pallas import tpu as pltpu
from jax.experimental.pallas import tpu_sc as plsc

N = 100000
K = 17
F = 8
NW = 32

SBA = 250
NSBA = N // SBA
MA = (NSBA + NW - 1) // NW

RB = 128
NBF = N // RB
TAIL = N - NBF * RB
NBT = NBF + 1
MB = (NBT + NW - 1) // NW
IDXB = RB * K
CH = IDXB // 128

_SC_PARAMS = pltpu.CompilerParams(
    use_tc_tiling_on_sc=False, needs_layout_passes=False)


def _sqrt16(x):
    xi = plsc.bitcast(x, jnp.int32)
    y = plsc.bitcast(jnp.int32(0x5F3759DF) - (xi >> 1), jnp.float32)
    for _ in range(3):
        y = y * (1.5 - 0.5 * x * y * y)
    return x * y


def _feat_body(tri_hbm, bary_hbm, feat_hbm, tri_v, bary_v, feat_v):
    wid = lax.axis_index("c") * 16 + lax.axis_index("s")

    def sub_batch(m, carry):
        b = m * NW + wid

        @pl.when(b < NSBA)
        def _():
            r0 = b * SBA
            pltpu.sync_copy(tri_hbm.at[pl.ds(r0, SBA)], tri_v)
            pltpu.sync_copy(bary_hbm.at[pl.ds(r0, SBA)], bary_v)
            lane = lax.iota(jnp.int32, 16)
            for g in range(16):
                mask = None if g < 15 else lane < (SBA - 15 * 16)
                rows = g * 16 + lane
                vg = [plsc.load_gather(
                          tri_v, [rows, jnp.full((16,), c // 3, jnp.int32),
                                  jnp.full((16,), c % 3, jnp.int32)],
                          mask=mask)
                      for c in range(9)]
                s = []
                for (a, b2) in ((0, 3), (0, 6), (3, 6)):
                    dx = vg[a] - vg[b2]
                    dy = vg[a + 1] - vg[b2 + 1]
                    dz = vg[a + 2] - vg[b2 + 2]
                    s.append(dx * dx + dy * dy + dz * dz)
                mn = _sqrt16(jnp.minimum(jnp.minimum(s[0], s[1]), s[2]))
                mx = _sqrt16(jnp.maximum(jnp.maximum(s[0], s[1]), s[2]))
                cols = [mn, mx]
                for c in range(3):
                    cols.append(plsc.load_gather(
                        bary_v, [rows, jnp.full((16,), c, jnp.int32)],
                        mask=mask))
                for c in range(5):
                    plsc.store_scatter(
                        feat_v, [rows, jnp.full((16,), c, jnp.int32)],
                        cols[c], mask=mask)
            pltpu.sync_copy(feat_v, feat_hbm.at[pl.ds(r0, SBA)])
        return carry

    lax.fori_loop(0, MA, sub_batch, 0)


def _feat_sc(tri, bary):
    mesh = plsc.VectorSubcoreMesh(core_axis_name="c", subcore_axis_name="s")
    return pl.kernel(
        _feat_body,
        out_type=jax.ShapeDtypeStruct((N, F), jnp.float32),
        mesh=mesh,
        scratch_types=[
            pltpu.VMEM((SBA, 3, 3), jnp.float32),
            pltpu.VMEM((SBA, 3), jnp.float32),
            pltpu.VMEM((SBA, F), jnp.float32),
        ],
        compiler_params=_SC_PARAMS,
    )(tri, bary)


def _flat_body(idx_hbm, idxf_hbm, idx2d_v, idxf_v):
    wid = lax.axis_index("c") * 16 + lax.axis_index("s")

    def batch(m, carry):
        b = m * NW + wid

        @pl.when(b < NBT)
        def _():
            @pl.when(b < NBF)
            def _():
                pltpu.sync_copy(idx_hbm.at[pl.ds(b * RB, RB)], idx2d_v)

            @pl.when(b == NBF)
            def _():
                pltpu.sync_copy(idx_hbm.at[pl.ds(NBF * RB, TAIL)],
                                idx2d_v.at[pl.ds(0, TAIL)])
            lane = lax.iota(jnp.int32, 16)
            for u in range(IDXB // 16):
                q0 = u * 16
                col = (q0 % K) + lane
                over = (col >= K).astype(jnp.int32)
                row = (q0 // K) + over
                col = col - K * over
                idxf_v[pl.ds(q0, 16)] = plsc.load_gather(idx2d_v, [row, col])

            @pl.when(b < NBF)
            def _():
                pltpu.sync_copy(idxf_v, idxf_hbm.at[pl.ds(b * IDXB, IDXB)])

            @pl.when(b == NBF)
            def _():
                pltpu.sync_copy(idxf_v.at[pl.ds(0, TAIL * K)],
                                idxf_hbm.at[pl.ds(NBF * IDXB, TAIL * K)])
        return carry

    lax.fori_loop(0, MB, batch, 0)


def _flatten_idx(idx):
    mesh = plsc.VectorSubcoreMesh(core_axis_name="c", subcore_axis_name="s")
    return pl.kernel(
        _flat_body,
        out_type=jax.ShapeDtypeStruct((N * K,), jnp.int32),
        mesh=mesh,
        scratch_types=[
            pltpu.VMEM((RB, K), jnp.int32),
            pltpu.VMEM((IDXB,), jnp.int32),
        ],
        compiler_params=pltpu.CompilerParams(
            use_tc_tiling_on_sc=True, needs_layout_passes=False),
    )(idx)


def _rmat_body(feat_hbm, idxf_hbm, out_hbm, idx0, idx1, g0, g1, o_v, gs0, gs1):
    wid = lax.axis_index("c") * 16 + lax.axis_index("s")

    def b_of(t):
        return t * NW + wid

    def stage_and_fire(b, idx_v, g_v, gsem):
        @pl.when(b < NBF)
        def _():
            pltpu.sync_copy(idxf_hbm.at[pl.ds(b * IDXB, IDXB)], idx_v)

        @pl.when(b == NBF)
        def _():
            pltpu.sync_copy(idxf_hbm.at[pl.ds(NBF * IDXB, TAIL * K)],
                            idx_v.at[pl.ds(0, TAIL * K)])
        for c in range(CH):
            pltpu.async_copy(feat_hbm.at[idx_v.at[pl.ds(c * 128, 128)]],
                             g_v.at[pl.ds(c * 128, 128)], gsem)

    def compute(b, idx_v, g_v, gsem):
        del idx_v
        pltpu.make_async_copy(feat_hbm.at[pl.ds(0, IDXB)], g_v, gsem).wait()
        nrows = jnp.where(b == NBF, TAIL, RB)

        def row_body(i, carry3):
            lane = lax.iota(jnp.int32, 16)
            bvec = jnp.full((16,), i * K, jnp.int32)
            for t in range(5):
                p = t * 16 + lane
                jr = p // 5
                jc = p % 5
                cvals = plsc.load_gather(g_v, [bvec, jc])
                nvals = plsc.load_gather(g_v, [bvec + 1 + jr, jc])
                o_v[i, pl.ds(t * 16, 16)] = cvals - nvals
            return carry3
        lax.fori_loop(0, nrows, row_body, 0)

        @pl.when(b < NBF)
        def _():
            pltpu.sync_copy(o_v, out_hbm.at[pl.ds(b * RB, RB)])

        @pl.when(b == NBF)
        def _():
            pltpu.sync_copy(o_v.at[pl.ds(0, TAIL)],
                            out_hbm.at[pl.ds(NBF * RB, TAIL)])

    def guarded(t, fn, *args):
        @pl.when(b_of(t) < NBT)
        def _():
            fn(b_of(t), *args)

    guarded(0, stage_and_fire, idx0, g0, gs0)

    def pair_body(q, carry):
        tA = 2 * q
        guarded(tA + 1, stage_and_fire, idx1, g1, gs1)
        guarded(tA, compute, idx0, g0, gs0)
        guarded(tA + 2, stage_and_fire, idx0, g0, gs0)
        guarded(tA + 1, compute, idx1, g1, gs1)
        return carry

    lax.fori_loop(0, (MB + 1) // 2, pair_body, 0)


def _rmatrix_sc(feat, idx):
    mesh = plsc.VectorSubcoreMesh(core_axis_name="c", subcore_axis_name="s")
    return pl.kernel(
        _rmat_body,
        out_type=jax.ShapeDtypeStruct((N, (K - 1) * 5), jnp.float32),
        mesh=mesh,
        scratch_types=[
            pltpu.VMEM((IDXB,), jnp.int32),
            pltpu.VMEM((IDXB,), jnp.int32),
            pltpu.VMEM((IDXB, F), jnp.float32),
            pltpu.VMEM((IDXB, F), jnp.float32),
            pltpu.VMEM((RB, (K - 1) * 5), jnp.float32),
            pltpu.SemaphoreType.DMA,
            pltpu.SemaphoreType.DMA,
        ],
        compiler_params=_SC_PARAMS,
    )(feat, idx)


def kernel(triangles, barycenters, indices_neigh_tri, number_neigh_tri):
    del number_neigh_tri
    idx32 = indices_neigh_tri.astype(jnp.int32)
    idxf = _flatten_idx(idx32)
    feat = _feat_sc(triangles, barycenters)
    out80 = _rmatrix_sc(feat, idxf)
    return out80.reshape(N, K - 1, 5)


# speedup vs baseline: 1.2536x; 1.2536x over previous
import jax
import jax.numpy as jnp
from jax import lax
from jax.experimental import pallas as pl
from jax.experimental.pallas import tpu as pltpu
from jax.experimental.pallas import tpu_sc as plsc

N = 100000
K = 17
F = 8
NW = 32

SBA = 250
NSBA = N // SBA
MA = (NSBA + NW - 1) // NW

RB = 128
NBF = N // RB
TAIL = N - NBF * RB
NBT = NBF + 1
MB = (NBT + NW - 1) // NW
IDXB = RB * K
CH = IDXB // 128

_SC_PARAMS = pltpu.CompilerParams(
    use_tc_tiling_on_sc=False, needs_layout_passes=False)


def _sqrt16(x):
    xi = plsc.bitcast(x, jnp.int32)
    y = plsc.bitcast(jnp.int32(0x5F3759DF) - (xi >> 1), jnp.float32)
    for _ in range(3):
        y = y * (1.5 - 0.5 * x * y * y)
    return x * y


def _feat_body(tri_hbm, bary_hbm, feat_hbm, tri_v, bary_v, feat_v):
    wid = lax.axis_index("c") * 16 + lax.axis_index("s")

    def sub_batch(m, carry):
        b = m * NW + wid

        @pl.when(b < NSBA)
        def _():
            r0 = b * SBA
            pltpu.sync_copy(tri_hbm.at[pl.ds(r0, SBA)], tri_v)
            pltpu.sync_copy(bary_hbm.at[pl.ds(r0, SBA)], bary_v)
            lane = lax.iota(jnp.int32, 16)
            for g in range(16):
                mask = None if g < 15 else lane < (SBA - 15 * 16)
                rows = g * 16 + lane
                vg = [plsc.load_gather(
                          tri_v, [rows, jnp.full((16,), c // 3, jnp.int32),
                                  jnp.full((16,), c % 3, jnp.int32)],
                          mask=mask)
                      for c in range(9)]
                s = []
                for (a, b2) in ((0, 3), (0, 6), (3, 6)):
                    dx = vg[a] - vg[b2]
                    dy = vg[a + 1] - vg[b2 + 1]
                    dz = vg[a + 2] - vg[b2 + 2]
                    s.append(dx * dx + dy * dy + dz * dz)
                mn = _sqrt16(jnp.minimum(jnp.minimum(s[0], s[1]), s[2]))
                mx = _sqrt16(jnp.maximum(jnp.maximum(s[0], s[1]), s[2]))
                cols = [mn, mx]
                for c in range(3):
                    cols.append(plsc.load_gather(
                        bary_v, [rows, jnp.full((16,), c, jnp.int32)],
                        mask=mask))
                for c in range(5):
                    plsc.store_scatter(
                        feat_v, [rows, jnp.full((16,), c, jnp.int32)],
                        cols[c], mask=mask)
            pltpu.sync_copy(feat_v, feat_hbm.at[pl.ds(r0, SBA)])
        return carry

    lax.fori_loop(0, MA, sub_batch, 0)


def _feat_sc(tri, bary):
    mesh = plsc.VectorSubcoreMesh(core_axis_name="c", subcore_axis_name="s")
    return pl.kernel(
        _feat_body,
        out_type=jax.ShapeDtypeStruct((N, F), jnp.float32),
        mesh=mesh,
        scratch_types=[
            pltpu.VMEM((SBA, 3, 3), jnp.float32),
            pltpu.VMEM((SBA, 3), jnp.float32),
            pltpu.VMEM((SBA, F), jnp.float32),
        ],
        compiler_params=_SC_PARAMS,
    )(tri, bary)


def _rmat_body(feat_hbm, idxt_hbm, out_hbm, idx0, idx1, g0, g1, o_v, gs0, gs1):
    wid = lax.axis_index("c") * 16 + lax.axis_index("s")

    def b_of(t):
        return t * NW + wid

    def stage_and_fire(b, idx_v, g_v, gsem):
        @pl.when(b < NBF)
        def _():
            pltpu.sync_copy(idxt_hbm.at[:, pl.ds(b * RB, RB)], idx_v)

        @pl.when(b == NBF)
        def _():
            pltpu.sync_copy(idxt_hbm.at[:, pl.ds(NBF * RB, TAIL)],
                            idx_v.at[:, pl.ds(0, TAIL)])
        for c in range(CH):
            pltpu.async_copy(feat_hbm.at[idx_v.at[c]],
                             g_v.at[pl.ds(c * RB, RB)], gsem)

    def compute(b, idx_v, g_v, gsem):
        del idx_v
        pltpu.make_async_copy(feat_hbm.at[pl.ds(0, IDXB)], g_v, gsem).wait()
        ngroups = jnp.where(b == NBF, TAIL // 16, RB // 16)

        def grp_body(g, carry3):
            rvec = g * 16 + lax.iota(jnp.int32, 16)
            for k in range(5):
                kvec = jnp.full((16,), k, jnp.int32)
                cvals = plsc.load_gather(g_v, [rvec, kvec])
                for j in range(16):
                    nvals = plsc.load_gather(
                        g_v, [(j + 1) * RB + rvec, kvec])
                    o_v[k, j, pl.ds(g * 16, 16)] = cvals - nvals
            return carry3
        lax.fori_loop(0, ngroups, grp_body, 0)

        @pl.when(b < NBF)
        def _():
            pltpu.sync_copy(o_v, out_hbm.at[:, :, pl.ds(b * RB, RB)])

        @pl.when(b == NBF)
        def _():
            pltpu.sync_copy(o_v.at[:, :, pl.ds(0, TAIL)],
                            out_hbm.at[:, :, pl.ds(NBF * RB, TAIL)])

    def guarded(t, fn, *args):
        @pl.when(b_of(t) < NBT)
        def _():
            fn(b_of(t), *args)

    guarded(0, stage_and_fire, idx0, g0, gs0)

    def pair_body(q, carry):
        tA = 2 * q
        guarded(tA + 1, stage_and_fire, idx1, g1, gs1)
        guarded(tA, compute, idx0, g0, gs0)
        guarded(tA + 2, stage_and_fire, idx0, g0, gs0)
        guarded(tA + 1, compute, idx1, g1, gs1)
        return carry

    lax.fori_loop(0, (MB + 1) // 2, pair_body, 0)


def _rmatrix_sc(feat, idx):
    mesh = plsc.VectorSubcoreMesh(core_axis_name="c", subcore_axis_name="s")
    return pl.kernel(
        _rmat_body,
        out_type=jax.ShapeDtypeStruct((5, K - 1, N), jnp.float32),
        mesh=mesh,
        scratch_types=[
            pltpu.VMEM((K, RB), jnp.int32),
            pltpu.VMEM((K, RB), jnp.int32),
            pltpu.VMEM((IDXB, F), jnp.float32),
            pltpu.VMEM((IDXB, F), jnp.float32),
            pltpu.VMEM((5, K - 1, RB), jnp.float32),
            pltpu.SemaphoreType.DMA,
            pltpu.SemaphoreType.DMA,
        ],
        compiler_params=_SC_PARAMS,
    )(feat, idx)


def kernel(triangles, barycenters, indices_neigh_tri, number_neigh_tri):
    del number_neigh_tri
    idx32 = indices_neigh_tri.astype(jnp.int32)
    feat = _feat_sc(triangles, barycenters)
    out_t = _rmatrix_sc(feat, idx32.T)
    return out_t.transpose(2, 1, 0)


# speedup vs baseline: 3.6203x; 2.8880x over previous
import jax
import jax.numpy as jnp
from jax import lax
from jax.experimental import pallas as pl
from jax.experimental.pallas import tpu as pltpu
from jax.experimental.pallas import tpu_sc as plsc

N = 100000
K = 17
F = 8
NW = 32

SBA = 160
NSBA = N // SBA
MA = (NSBA + NW - 1) // NW

RB = 128
NBF = N // RB
TAIL = N - NBF * RB
NBT = NBF + 1
MB = (NBT + NW - 1) // NW
IDXB = RB * K
CH = IDXB // 128

_SC_PARAMS = pltpu.CompilerParams(
    use_tc_tiling_on_sc=False, needs_layout_passes=False)


def _sqrt16(x):
    xi = plsc.bitcast(x, jnp.int32)
    y = plsc.bitcast(jnp.int32(0x5F3759DF) - (xi >> 1), jnp.float32)
    for _ in range(3):
        y = y * (1.5 - 0.5 * x * y * y)
    return x * y


def _feat_body(tri_hbm, bary_hbm, feat_hbm, tri_v, bary_v, feat_v):
    wid = lax.axis_index("c") * 16 + lax.axis_index("s")

    def sub_batch(m, carry):
        b = m * NW + wid

        @pl.when(b < NSBA)
        def _():
            r0 = b * SBA
            pltpu.sync_copy(tri_hbm.at[:, :, pl.ds(r0, SBA)], tri_v)
            pltpu.sync_copy(bary_hbm.at[:, pl.ds(r0, SBA)], bary_v)
            lane = lax.iota(jnp.int32, 16)
            for g in range(SBA // 16):
                rows = g * 16 + lane
                sl = pl.ds(g * 16, 16)
                vg = [tri_v[c // 3, c % 3, sl] for c in range(9)]
                s = []
                for (a, b2) in ((0, 3), (0, 6), (3, 6)):
                    dx = vg[a] - vg[b2]
                    dy = vg[a + 1] - vg[b2 + 1]
                    dz = vg[a + 2] - vg[b2 + 2]
                    s.append(dx * dx + dy * dy + dz * dz)
                mn = _sqrt16(jnp.minimum(jnp.minimum(s[0], s[1]), s[2]))
                mx = _sqrt16(jnp.maximum(jnp.maximum(s[0], s[1]), s[2]))
                cols = [mn, mx, bary_v[0, sl], bary_v[1, sl], bary_v[2, sl]]
                for c in range(5):
                    plsc.store_scatter(
                        feat_v, [rows, jnp.full((16,), c, jnp.int32)],
                        cols[c])
            pltpu.sync_copy(feat_v, feat_hbm.at[pl.ds(r0, SBA)])
        return carry

    lax.fori_loop(0, MA, sub_batch, 0)


def _feat_sc(tri_t, bary_t):
    mesh = plsc.VectorSubcoreMesh(core_axis_name="c", subcore_axis_name="s")
    return pl.kernel(
        _feat_body,
        out_type=jax.ShapeDtypeStruct((N, F), jnp.float32),
        mesh=mesh,
        scratch_types=[
            pltpu.VMEM((3, 3, SBA), jnp.float32),
            pltpu.VMEM((3, SBA), jnp.float32),
            pltpu.VMEM((SBA, F), jnp.float32),
        ],
        compiler_params=_SC_PARAMS,
    )(tri_t, bary_t)


N2 = NBT * RB


def _shuf_body(idxt_hbm, out_hbm, iv0, iv1, dv, ss0, ss1, ws0, ws1):
    wid = lax.axis_index("c") * 16 + lax.axis_index("s")

    def b_of(t):
        return t * NW + wid

    def stage(b, idx_v, ssem):
        pltpu.async_copy(idxt_hbm.at[:, pl.ds(b * RB, RB)], idx_v, ssem)

    def process(b, idx_v, ssem, wsem):
        pltpu.make_async_copy(idxt_hbm.at[:, pl.ds(b * RB, RB)],
                              idx_v, ssem).wait()
        for c in range(K):
            pltpu.async_copy(idx_v.at[c],
                             out_hbm.at[pl.ds(c * N2 + b * RB, RB)], wsem)

    def drain_w(b, wsem):
        del b
        pltpu.make_async_copy(out_hbm.at[pl.ds(0, K * RB)], dv, wsem).wait()

    def guarded(t, fn, *args):
        @pl.when(b_of(t) < NBT)
        def _():
            fn(b_of(t), *args)

    guarded(0, stage, iv0, ss0)
    guarded(1, stage, iv1, ss1)

    def pair_body(q, carry):
        tA = 2 * q
        guarded(tA, process, iv0, ss0, ws0)
        guarded(tA + 1, process, iv1, ss1, ws1)

        def reuse0(b):
            drain_w(b, ws0)
            stage(b, iv0, ss0)

        def reuse1(b):
            drain_w(b, ws1)
            stage(b, iv1, ss1)
        guarded(tA + 2, reuse0)
        guarded(tA + 3, reuse1)
        return carry

    lax.fori_loop(0, (MB + 1) // 2, pair_body, 0)
    drain_w(0, ws0)
    drain_w(0, ws1)


def _shuffle_idx(idx_t):
    mesh = plsc.VectorSubcoreMesh(core_axis_name="c", subcore_axis_name="s")
    return pl.kernel(
        _shuf_body,
        out_type=jax.ShapeDtypeStruct((K * N2,), jnp.int32),
        mesh=mesh,
        scratch_types=[
            pltpu.VMEM((K, RB), jnp.int32),
            pltpu.VMEM((K, RB), jnp.int32),
            pltpu.VMEM((K * RB,), jnp.int32),
            pltpu.SemaphoreType.DMA,
            pltpu.SemaphoreType.DMA,
            pltpu.SemaphoreType.DMA,
            pltpu.SemaphoreType.DMA,
        ],
        compiler_params=pltpu.CompilerParams(
            use_tc_tiling_on_sc=True, needs_layout_passes=False),
    )(idx_t)


def _rmat_body(feat_hbm, idxt_hbm, out_hbm, idx0, idx1, g0, g1, o_v, gs0, gs1):
    wid = lax.axis_index("c") * 16 + lax.axis_index("s")

    def b_of(t):
        return t * NW + wid

    def stage_and_fire(b, idx_v, g_v, gsem):
        @pl.when(b < NBF)
        def _():
            pltpu.sync_copy(idxt_hbm.at[:, pl.ds(b * RB, RB)], idx_v)

        @pl.when(b == NBF)
        def _():
            pltpu.sync_copy(idxt_hbm.at[:, pl.ds(NBF * RB, TAIL)],
                            idx_v.at[:, pl.ds(0, TAIL)])
        for c in range(CH):
            pltpu.async_copy(feat_hbm.at[idx_v.at[c]],
                             g_v.at[pl.ds(c * RB, RB)], gsem)

    def compute(b, idx_v, g_v, gsem):
        del idx_v
        pltpu.make_async_copy(feat_hbm.at[pl.ds(0, IDXB)], g_v, gsem).wait()
        ngroups = jnp.where(b == NBF, TAIL // 16, RB // 16)

        def grp_body(g, carry3):
            rvec = g * 16 + lax.iota(jnp.int32, 16)
            for k in range(5):
                kvec = jnp.full((16,), k, jnp.int32)
                cvals = plsc.load_gather(g_v, [rvec, kvec])
                for j in range(16):
                    nvals = plsc.load_gather(
                        g_v, [(j + 1) * RB + rvec, kvec])
                    o_v[k, j, pl.ds(g * 16, 16)] = cvals - nvals
            return carry3
        lax.fori_loop(0, ngroups, grp_body, 0)

        @pl.when(b < NBF)
        def _():
            pltpu.sync_copy(o_v, out_hbm.at[:, :, pl.ds(b * RB, RB)])

        @pl.when(b == NBF)
        def _():
            pltpu.sync_copy(o_v.at[:, :, pl.ds(0, TAIL)],
                            out_hbm.at[:, :, pl.ds(NBF * RB, TAIL)])

    def guarded(t, fn, *args):
        @pl.when(b_of(t) < NBT)
        def _():
            fn(b_of(t), *args)

    guarded(0, stage_and_fire, idx0, g0, gs0)

    def pair_body(q, carry):
        tA = 2 * q
        guarded(tA + 1, stage_and_fire, idx1, g1, gs1)
        guarded(tA, compute, idx0, g0, gs0)
        guarded(tA + 2, stage_and_fire, idx0, g0, gs0)
        guarded(tA + 1, compute, idx1, g1, gs1)
        return carry

    lax.fori_loop(0, (MB + 1) // 2, pair_body, 0)


def _rmatrix_sc(feat, idx):
    mesh = plsc.VectorSubcoreMesh(core_axis_name="c", subcore_axis_name="s")
    return pl.kernel(
        _rmat_body,
        out_type=jax.ShapeDtypeStruct((5, K - 1, N), jnp.float32),
        mesh=mesh,
        scratch_types=[
            pltpu.VMEM((K, RB), jnp.int32),
            pltpu.VMEM((K, RB), jnp.int32),
            pltpu.VMEM((IDXB, F), jnp.float32),
            pltpu.VMEM((IDXB, F), jnp.float32),
            pltpu.VMEM((5, K - 1, RB), jnp.float32),
            pltpu.SemaphoreType.DMA,
            pltpu.SemaphoreType.DMA,
        ],
        name="rmat_gather_diff",
        compiler_params=_SC_PARAMS,
    )(feat, idx)


def kernel(triangles, barycenters, indices_neigh_tri, number_neigh_tri):
    del number_neigh_tri
    idx32 = indices_neigh_tri.astype(jnp.int32)
    idxs = _shuffle_idx(idx32.T).reshape(K, N2)
    feat = _feat_sc(triangles.transpose(1, 2, 0), barycenters.T)
    out_t = _rmatrix_sc(feat, idxs)
    return out_t.transpose(2, 1, 0)


# speedup vs baseline: 3.7446x; 1.0343x over previous
import jax
import jax.numpy as jnp
from jax import lax
from jax.experimental import pallas as pl
from jax.experimental.pallas import tpu as pltpu
from jax.experimental.pallas import tpu_sc as plsc

N = 100000
K = 17
F = 8
NW = 32

SBA = 160
NSBA = N // SBA
MA = (NSBA + NW - 1) // NW

RB = 128
NBF = N // RB
TAIL = N - NBF * RB
NBT = NBF + 1
MB = (NBT + NW - 1) // NW
IDXB = RB * K
CH = IDXB // 128

_SC_PARAMS = pltpu.CompilerParams(
    use_tc_tiling_on_sc=False, needs_layout_passes=False)


def _sqrt16(x):
    xi = plsc.bitcast(x, jnp.int32)
    y = plsc.bitcast(jnp.int32(0x5F3759DF) - (xi >> 1), jnp.float32)
    for _ in range(3):
        y = y * (1.5 - 0.5 * x * y * y)
    return x * y


def _feat_body(tri_hbm, bary_hbm, feat_hbm, tri_v, bary_v, feat_v):
    wid = lax.axis_index("c") * 16 + lax.axis_index("s")

    def sub_batch(m, carry):
        b = m * NW + wid

        @pl.when(b < NSBA)
        def _():
            r0 = b * SBA
            pltpu.sync_copy(tri_hbm.at[:, :, pl.ds(r0, SBA)], tri_v)
            pltpu.sync_copy(bary_hbm.at[:, pl.ds(r0, SBA)], bary_v)
            lane = lax.iota(jnp.int32, 16)
            for g in range(SBA // 16):
                rows = g * 16 + lane
                sl = pl.ds(g * 16, 16)
                vg = [tri_v[c // 3, c % 3, sl] for c in range(9)]
                s = []
                for (a, b2) in ((0, 3), (0, 6), (3, 6)):
                    dx = vg[a] - vg[b2]
                    dy = vg[a + 1] - vg[b2 + 1]
                    dz = vg[a + 2] - vg[b2 + 2]
                    s.append(dx * dx + dy * dy + dz * dz)
                mn = _sqrt16(jnp.minimum(jnp.minimum(s[0], s[1]), s[2]))
                mx = _sqrt16(jnp.maximum(jnp.maximum(s[0], s[1]), s[2]))
                cols = [mn, mx, bary_v[0, sl], bary_v[1, sl], bary_v[2, sl]]
                for c in range(5):
                    plsc.store_scatter(
                        feat_v, [rows, jnp.full((16,), c, jnp.int32)],
                        cols[c])
            pltpu.sync_copy(feat_v, feat_hbm.at[pl.ds(r0, SBA)])
        return carry

    lax.fori_loop(0, MA, sub_batch, 0)


def _feat_sc(tri_t, bary_t):
    mesh = plsc.VectorSubcoreMesh(core_axis_name="c", subcore_axis_name="s")
    return pl.kernel(
        _feat_body,
        out_type=jax.ShapeDtypeStruct((N, F), jnp.float32),
        mesh=mesh,
        scratch_types=[
            pltpu.VMEM((3, 3, SBA), jnp.float32),
            pltpu.VMEM((3, SBA), jnp.float32),
            pltpu.VMEM((SBA, F), jnp.float32),
        ],
        compiler_params=_SC_PARAMS,
    )(tri_t, bary_t)


N2 = NBT * RB


def _shuf_body(idxt_hbm, out_hbm, iv0, iv1, dv, ss0, ss1, ws0, ws1):
    wid = lax.axis_index("c") * 16 + lax.axis_index("s")

    def b_of(t):
        return t * NW + wid

    def stage(b, idx_v, ssem):
        pltpu.async_copy(idxt_hbm.at[:, pl.ds(b * RB, RB)], idx_v, ssem)

    def process(b, idx_v, ssem, wsem):
        pltpu.make_async_copy(idxt_hbm.at[:, pl.ds(b * RB, RB)],
                              idx_v, ssem).wait()
        for c in range(K):
            pltpu.async_copy(idx_v.at[c],
                             out_hbm.at[pl.ds(c * N2 + b * RB, RB)], wsem)

    def drain_w(b, wsem):
        del b
        pltpu.make_async_copy(out_hbm.at[pl.ds(0, K * RB)], dv, wsem).wait()

    def guarded(t, fn, *args):
        @pl.when(b_of(t) < NBT)
        def _():
            fn(b_of(t), *args)

    guarded(0, stage, iv0, ss0)
    guarded(1, stage, iv1, ss1)

    def pair_body(q, carry):
        tA = 2 * q
        guarded(tA, process, iv0, ss0, ws0)
        guarded(tA + 1, process, iv1, ss1, ws1)

        def reuse0(b):
            drain_w(b, ws0)
            stage(b, iv0, ss0)

        def reuse1(b):
            drain_w(b, ws1)
            stage(b, iv1, ss1)
        guarded(tA + 2, reuse0)
        guarded(tA + 3, reuse1)
        return carry

    lax.fori_loop(0, (MB + 1) // 2, pair_body, 0)
    drain_w(0, ws0)
    drain_w(0, ws1)


def _shuffle_idx(idx_t):
    mesh = plsc.VectorSubcoreMesh(core_axis_name="c", subcore_axis_name="s")
    return pl.kernel(
        _shuf_body,
        out_type=jax.ShapeDtypeStruct((K * N2,), jnp.int32),
        mesh=mesh,
        scratch_types=[
            pltpu.VMEM((K, RB), jnp.int32),
            pltpu.VMEM((K, RB), jnp.int32),
            pltpu.VMEM((K * RB,), jnp.int32),
            pltpu.SemaphoreType.DMA,
            pltpu.SemaphoreType.DMA,
            pltpu.SemaphoreType.DMA,
            pltpu.SemaphoreType.DMA,
        ],
        compiler_params=pltpu.CompilerParams(
            use_tc_tiling_on_sc=True, needs_layout_passes=False),
    )(idx_t)


def _rmat_body(feat_hbm, idxt_hbm, out_hbm,
               idx0, idx1, g0, g1, o0, o1, gs0, gs1, os0, os1):
    wid = lax.axis_index("c") * 16 + lax.axis_index("s")

    def b_of(t):
        return t * NW + wid

    def stage_and_fire(b, idx_v, g_v, gsem):
        @pl.when(b < NBF)
        def _():
            pltpu.sync_copy(idxt_hbm.at[:, pl.ds(b * RB, RB)], idx_v)

        @pl.when(b == NBF)
        def _():
            pltpu.sync_copy(idxt_hbm.at[:, pl.ds(NBF * RB, TAIL)],
                            idx_v.at[:, pl.ds(0, TAIL)])
        for c in range(CH):
            pltpu.async_copy(feat_hbm.at[idx_v.at[c]],
                             g_v.at[pl.ds(c * RB, RB)], gsem)

    def compute(b, idx_v, g_v, gsem, o_v, osem, first):
        del idx_v
        pltpu.make_async_copy(feat_hbm.at[pl.ds(0, IDXB)], g_v, gsem).wait()

        @pl.when(jnp.logical_not(first))
        def _():
            pltpu.make_async_copy(out_hbm.at[:, :, pl.ds(0, RB)],
                                  o_v, osem).wait()
        ngroups = jnp.where(b == NBF, TAIL // 16, RB // 16)

        def grp_body(g, carry3):
            rvec = g * 16 + lax.iota(jnp.int32, 16)
            for k in range(5):
                kvec = jnp.full((16,), k, jnp.int32)
                cvals = plsc.load_gather(g_v, [rvec, kvec])
                for j in range(16):
                    nvals = plsc.load_gather(
                        g_v, [(j + 1) * RB + rvec, kvec])
                    o_v[k, j, pl.ds(g * 16, 16)] = cvals - nvals
            return carry3
        lax.fori_loop(0, ngroups, grp_body, 0)

        @pl.when(b < NBF)
        def _():
            pltpu.async_copy(o_v, out_hbm.at[:, :, pl.ds(b * RB, RB)], osem)

        @pl.when(b == NBF)
        def _():
            pltpu.sync_copy(o_v.at[:, :, pl.ds(0, TAIL)],
                            out_hbm.at[:, :, pl.ds(NBF * RB, TAIL)])

    def guarded(t, fn, *args):
        @pl.when(b_of(t) < NBT)
        def _():
            fn(b_of(t), *args)

    guarded(0, stage_and_fire, idx0, g0, gs0)

    def pair_body(q, carry):
        tA = 2 * q
        guarded(tA + 1, stage_and_fire, idx1, g1, gs1)
        guarded(tA, compute, idx0, g0, gs0, o0, os0, q == 0)
        guarded(tA + 2, stage_and_fire, idx0, g0, gs0)
        guarded(tA + 1, compute, idx1, g1, gs1, o1, os1, q == 0)
        return carry

    lax.fori_loop(0, (MB + 1) // 2, pair_body, 0)

    @pl.when(b_of(MB - 1) != NBF)
    def _():
        pltpu.make_async_copy(out_hbm.at[:, :, pl.ds(0, RB)], o0, os0).wait()
    pltpu.make_async_copy(out_hbm.at[:, :, pl.ds(0, RB)], o1, os1).wait()


def _rmatrix_sc(feat, idx):
    mesh = plsc.VectorSubcoreMesh(core_axis_name="c", subcore_axis_name="s")
    return pl.kernel(
        _rmat_body,
        out_type=jax.ShapeDtypeStruct((5, K - 1, N), jnp.float32),
        mesh=mesh,
        scratch_types=[
            pltpu.VMEM((K, RB), jnp.int32),
            pltpu.VMEM((K, RB), jnp.int32),
            pltpu.VMEM((IDXB, F), jnp.float32),
            pltpu.VMEM((IDXB, F), jnp.float32),
            pltpu.VMEM((5, K - 1, RB), jnp.float32),
            pltpu.VMEM((5, K - 1, RB), jnp.float32),
            pltpu.SemaphoreType.DMA,
            pltpu.SemaphoreType.DMA,
            pltpu.SemaphoreType.DMA,
            pltpu.SemaphoreType.DMA,
        ],
        name="rmat_gather_diff",
        compiler_params=_SC_PARAMS,
    )(feat, idx)


def kernel(triangles, barycenters, indices_neigh_tri, number_neigh_tri):
    del number_neigh_tri
    idx32 = indices_neigh_tri.astype(jnp.int32)
    idxs = _shuffle_idx(idx32.T).reshape(K, N2)
    feat = _feat_sc(triangles.transpose(1, 2, 0), barycenters.T)
    out_t = _rmatrix_sc(feat, idxs)
    return out_t.transpose(2, 1, 0)
